# baseline (speedup 1.0000x reference)
import functools

import jax
import jax.numpy as jnp
from jax import lax
from jax.experimental import pallas as pl
from jax.experimental.pallas import tpu as pltpu
from jax.experimental.pallas import tpu_sc as plsc

_B = 16384
_D = 80
_NW = 32
_BPW = _B // _NW
_CHUNK = 128
_NCH = _BPW // _CHUNK

_SMALL = [
    (0, 0, 8, 0),
    (1, 8, 8, 23),
    (2, 16, 4, 40),
    (3, 20, 4, 46),
    (4, 24, 8, 64),
    (5, 32, 8, 119),
]
_SMALL_WORDS = 153 * 8
_SMALL_PAD = 1280
_BIG = [
    (6, 40, 16),
    (7, 56, 8),
    (8, 64, 8),
    (9, 72, 8),
]
_NBIG = 100001
_CROWS = 128
_NBLK = 781
_LAST_OFF = 99840
_GPW = (_NBLK + _NW - 1) // _NW
_TAIL_ROW = 99968
_TROWS = 33
_L16 = 12504
_L8 = 6256


def _compact_body(*refs):
    tabs = refs[0:4]
    outs = refs[4:8]
    vb16 = refs[8:11]
    vb8 = refs[11:14]
    ob = refs[14:16]
    vbta = refs[16]
    vbtb = refs[17]
    sem_in = refs[18]
    sem_out = refs[19]

    wid = lax.axis_index("s") * 2 + lax.axis_index("c")
    iota = lax.iota(jnp.int32, 16)
    prow16 = iota >> 3
    pcol8 = iota & 7

    for t in range(4):
        w = 16 if t == 0 else 8
        vb = vb16 if t == 0 else vb8

        def off_of(g):
            return pl.multiple_of(
                jnp.minimum((wid + g * _NW) * _CROWS, _LAST_OFF), _CROWS)

        incp = [pltpu.async_copy(
            tabs[t].at[pl.ds(off_of(p), _CROWS), :], vb[p], sem_in)
            for p in range(3)]
        outcp = []
        for g in range(_GPW):
            incp[g].wait()
            if g >= 2:
                outcp[g - 2].wait()
            src = vb[g % 3]
            dst = ob[g % 2]

            if w == 16:
                def cp16(j, carry):
                    dst[j >> 3, pl.ds((j & 7) * 16, 16)] = src[j, pl.ds(0, 16)]
                    return carry
                lax.fori_loop(0, _CROWS, cp16, 0, unroll=4)
            else:
                def cp8(j, carry):
                    v = plsc.load_gather(src, [prow16 + j * 2, pcol8])
                    dst[j >> 3, pl.ds((j & 7) * 16, 16)] = v
                    return carry
                lax.fori_loop(0, _CROWS // 2, cp8, 0, unroll=4)

            nlines = _CROWS * w // 128
            line0 = pl.multiple_of(off_of(g) * w // 128, 8)
            outcp.append(pltpu.async_copy(
                dst.at[pl.ds(0, nlines), :],
                outs[t].at[pl.ds(line0, nlines), :], sem_out))
            if g + 3 < _GPW:
                incp.append(pltpu.async_copy(
                    tabs[t].at[pl.ds(off_of(g + 3), _CROWS), :],
                    vb[(g + 3) % 3], sem_in))
        outcp[-2].wait()
        outcp[-1].wait()

    @pl.when(wid == 0)
    def _():
        for t in range(4):
            w = 16 if t == 0 else 8
            vbt = vbta if t == 0 else vbtb
            pltpu.sync_copy(tabs[t].at[pl.ds(_TAIL_ROW, _TROWS), :], vbt)
            if w == 16:
                def tcp16(j, carry):
                    ob[0][j >> 3, pl.ds((j & 7) * 16, 16)] = vbt[j, pl.ds(0, 16)]
                    return carry
                lax.fori_loop(0, _TROWS, tcp16, 0, unroll=2)
            else:
                def tcp8(j, carry):
                    v = plsc.load_gather(vbt, [prow16 + j * 2, pcol8])
                    ob[0][j >> 3, pl.ds((j & 7) * 16, 16)] = v
                    return carry
                lax.fori_loop(0, (_TROWS - 1) // 2, tcp8, 0, unroll=2)
                v = plsc.load_gather(vbt, [iota * 0 + (_TROWS - 1), pcol8])
                ob[0][2, pl.ds(0, 16)] = v
            pltpu.sync_copy(ob[0].at[pl.ds(0, 8), :],
                            outs[t].at[pl.ds(_TAIL_ROW * w // 128, 8), :])


def _gather_body(*refs):
    idx_hbm = refs[0]
    small_hbm = refs[1]
    ctabs = refs[2:6]
    out_hbm = refs[6]
    idx_v = refs[7]
    lidx = refs[8]
    small_v = refs[9]
    bufs = refs[10:14]
    stage = refs[14]
    sem = refs[15]

    wid = lax.axis_index("s") * 2 + lax.axis_index("c")
    pltpu.sync_copy(idx_hbm.at[wid], idx_v)
    pltpu.sync_copy(small_hbm, small_v)

    iota = lax.iota(jnp.int32, 16)
    izero = iota * 0
    prow16 = iota >> 3
    pcol8 = iota & 7

    for c in range(_NCH):
        base = wid * _BPW + c * _CHUNK

        def lineprep(g, carry):
            for k, (s, col, w) in enumerate(_BIG):
                iv = idx_v[s * _NCH + c, pl.ds(g * 16, 16)]
                lidx[k, pl.ds(g * 16, 16)] = iv >> (3 if w == 16 else 4)
            return carry

        lax.fori_loop(0, _CHUNK // 16, lineprep, 0, unroll=2)

        copies = [
            pltpu.async_copy(ctabs[k].at[lidx.at[k]], bufs[k], sem)
            for k in range(4)
        ]

        def small_group(g, carry):
            rows = iota + g * 16
            for s, col, w, off in _SMALL:
                idxv = idx_v[s * _NCH + c, pl.ds(g * 16, 16)]
                rowbase = (idxv << 3) + (off * 8)
                for k in range(w):
                    vals = plsc.load_gather(small_v, [rowbase + k])
                    plsc.store_scatter(stage, [rows, izero + (col + k)], vals)
            return carry

        lax.fori_loop(0, _CHUNK // 16, small_group, 0, unroll=False)

        for cp in copies:
            cp.wait()

        s16 = _BIG[0][0] * _NCH + c

        def row16(j, carry):
            jv = izero + j
            iv = plsc.load_gather(idx_v, [izero + s16, jv])
            colv = ((iv & 7) << 4) + iota
            v = plsc.load_gather(bufs[0], [jv, colv])
            plsc.store_scatter(stage, [jv, iota + 40], v)
            return carry

        lax.fori_loop(0, _CHUNK, row16, 0, unroll=4)

        def pair8(j, carry):
            rows = prow16 + (j * 2)
            for k, (s, col, w) in enumerate(_BIG[1:], start=1):
                iv = plsc.load_gather(idx_v, [izero + (s * _NCH + c), rows])
                colv = ((iv & 15) << 3) + pcol8
                v = plsc.load_gather(bufs[k], [rows, colv])
                plsc.store_scatter(stage, [rows, pcol8 + col], v)
            return carry

        lax.fori_loop(0, _CHUNK // 2, pair8, 0, unroll=2)

        pltpu.sync_copy(
            stage, out_hbm.at[pl.ds(pl.multiple_of(base, _CHUNK), _CHUNK), :])


@jax.jit
def kernel(city_idx, W_city, state_idx, W_state, store_type_idx, W_store_type,
           cluster_idx, W_cluster, store_nbr_idx, W_store_nbr, family_idx,
           W_family, store_family_interaction_idx, W_store_family_interaction,
           onpromo_promo_sum7_interaction_idx, W_onpromo_promo_sum7_interaction,
           onpromo_state_interaction_idx, W_onpromo_state_interaction,
           promo_sum7_state_interaction_idx, W_promo_sum7_state_interaction):
    idxs = [city_idx, state_idx, store_type_idx, cluster_idx, store_nbr_idx,
            family_idx, store_family_interaction_idx,
            onpromo_promo_sum7_interaction_idx, onpromo_state_interaction_idx,
            promo_sum7_state_interaction_idx]
    idxall = (jnp.stack(idxs, axis=0)
              .reshape(10, _NW, _NCH, _CHUNK)
              .transpose(1, 0, 2, 3)
              .reshape(_NW, 10 * _NCH, _CHUNK))
    smalls = [W_city, W_state, W_store_type, W_cluster, W_store_nbr, W_family]
    small_tab = jnp.concatenate(
        [jnp.pad(t, ((0, 0), (0, 8 - t.shape[1]))) for t in smalls], axis=0
    ).reshape(-1)
    small_tab = jnp.pad(small_tab, (0, _SMALL_PAD - _SMALL_WORDS))
    bigtabs = [W_store_family_interaction, W_onpromo_promo_sum7_interaction,
               W_onpromo_state_interaction, W_promo_sum7_state_interaction]

    mesh = plsc.VectorSubcoreMesh(core_axis_name="c", subcore_axis_name="s")
    tiled_params = pltpu.CompilerParams(
        use_tc_tiling_on_sc=True, needs_layout_passes=False)

    compact = functools.partial(
        pl.kernel,
        mesh=mesh,
        out_type=[jax.ShapeDtypeStruct((_L16, 128), jnp.float32)] +
                 [jax.ShapeDtypeStruct((_L8, 128), jnp.float32)] * 3,
        scratch_types=[
            pltpu.VMEM((_CROWS, 16), jnp.float32),
            pltpu.VMEM((_CROWS, 16), jnp.float32),
            pltpu.VMEM((_CROWS, 16), jnp.float32),
            pltpu.VMEM((_CROWS, 8), jnp.float32),
            pltpu.VMEM((_CROWS, 8), jnp.float32),
            pltpu.VMEM((_CROWS, 8), jnp.float32),
            pltpu.VMEM((16, 128), jnp.float32),
            pltpu.VMEM((16, 128), jnp.float32),
            pltpu.VMEM((_TROWS, 16), jnp.float32),
            pltpu.VMEM((_TROWS, 8), jnp.float32),
            pltpu.SemaphoreType.DMA,
            pltpu.SemaphoreType.DMA,
        ],
        compiler_params=tiled_params,
    )(_compact_body)
    ctabs = compact(*bigtabs)

    run = functools.partial(
        pl.kernel,
        mesh=mesh,
        out_type=jax.ShapeDtypeStruct((_B, _D), jnp.float32),
        scratch_types=[
            pltpu.VMEM((10 * _NCH, _CHUNK), jnp.int32),
            pltpu.VMEM((4, _CHUNK), jnp.int32),
            pltpu.VMEM((_SMALL_PAD,), jnp.float32),
            pltpu.VMEM((_CHUNK, 128), jnp.float32),
            pltpu.VMEM((_CHUNK, 128), jnp.float32),
            pltpu.VMEM((_CHUNK, 128), jnp.float32),
            pltpu.VMEM((_CHUNK, 128), jnp.float32),
            pltpu.VMEM((_CHUNK, _D), jnp.float32),
            pltpu.SemaphoreType.DMA,
        ],
        compiler_params=tiled_params,
    )(_gather_body)
    return run(idxall, small_tab, *ctabs)


# speedup vs baseline: 1.0495x; 1.0495x over previous
import functools

import jax
import jax.numpy as jnp
from jax import lax
from jax.experimental import pallas as pl
from jax.experimental.pallas import tpu as pltpu
from jax.experimental.pallas import tpu_sc as plsc

_B = 16384
_D = 80
_NW = 32
_BPW = _B // _NW
_CHUNK = 128
_NCH = _BPW // _CHUNK

_SMALL = [
    (0, 0, 8, 0),
    (1, 8, 8, 23),
    (2, 16, 4, 40),
    (3, 20, 4, 46),
    (4, 24, 8, 64),
    (5, 32, 8, 119),
]
_SMALL_WORDS = 153 * 8
_BIG = [
    (6, 40, 16),
    (7, 56, 8),
    (8, 64, 8),
    (9, 72, 8),
]
_NBIG = 100001
_CROWS = 128
_NBLK = 782
_TAIL_OFF = 99872
_GPW = (_NBLK + _NW - 1) // _NW
_NPAD = _NBIG + 1


def _compact_body(*refs):
    tabs = refs[0:4]
    outs = refs[4:8]
    vb16 = refs[8:11]
    vb8 = refs[11:14]
    ob = refs[14:16]
    vb1a = refs[16]
    vb1b = refs[17]
    sem_in = refs[18]
    sem_out = refs[19]

    wid = lax.axis_index("s") * 2 + lax.axis_index("c")
    iota = lax.iota(jnp.int32, 16)
    prow16 = iota >> 3
    pcol8 = iota & 7

    for t in range(4):
        w = 16 if t == 0 else 8
        vb = vb16 if t == 0 else vb8

        def off_of(g):
            return jnp.minimum((wid + g * _NW) * _CROWS, _TAIL_OFF)

        incp = [pltpu.async_copy(
            tabs[t].at[pl.ds(off_of(p), _CROWS), :], vb[p], sem_in)
            for p in range(3)]
        outcp = []
        for g in range(_GPW):
            incp[g].wait()
            if g >= 2:
                outcp[g - 2].wait()
            src = vb[g % 3]
            dst = ob[g % 2]

            if w == 16:
                def cp16(j, carry):
                    dst[pl.ds(j * 16, 16)] = src[j, pl.ds(0, 16)]
                    return carry
                lax.fori_loop(0, _CROWS, cp16, 0, unroll=4)
            else:
                def cp8(j, carry):
                    v = plsc.load_gather(src, [prow16 + j * 2, pcol8])
                    dst[pl.ds(j * 16, 16)] = v
                    return carry
                lax.fori_loop(0, _CROWS // 2, cp8, 0, unroll=4)

            outcp.append(pltpu.async_copy(
                dst.at[pl.ds(0, _CROWS * w)],
                outs[t].at[pl.ds(off_of(g) * w, _CROWS * w)], sem_out))
            if g + 3 < _GPW:
                incp.append(pltpu.async_copy(
                    tabs[t].at[pl.ds(off_of(g + 3), _CROWS), :],
                    vb[(g + 3) % 3], sem_in))
        outcp[-2].wait()
        outcp[-1].wait()

    @pl.when(wid == 0)
    def _():
        for t in range(4):
            w = 16 if t == 0 else 8
            vb1 = vb1a if t == 0 else vb1b
            pltpu.sync_copy(tabs[t].at[pl.ds(_NBIG - 1, 1), :], vb1)
            if w == 16:
                ob[0][pl.ds(0, 16)] = vb1a[0, :]
            else:
                v = plsc.load_gather(vb1b, [iota * 0, pcol8])
                ob[0][pl.ds(0, 16)] = v
            pltpu.sync_copy(ob[0].at[pl.ds(0, w)],
                            outs[t].at[pl.ds((_NBIG - 1) * w, w)])


def _gather_body(*refs):
    idx_hbm = refs[0:10]
    small_hbm = refs[10]
    bigtabs = refs[11:15]
    out_hbm = refs[15]
    idx_v = refs[16]
    small_v = refs[17]
    bufs = refs[18:22]
    stage = refs[22]
    sem = refs[23]

    wid = lax.axis_index("s") * 2 + lax.axis_index("c")
    for f in range(10):
        pltpu.sync_copy(idx_hbm[f].at[wid], idx_v.at[f])
    pltpu.sync_copy(small_hbm, small_v)

    iota = lax.iota(jnp.int32, 16)
    rowpat = iota * _D
    pat8 = (iota >> 3) * _D + (iota & 7)
    prow16 = iota >> 3
    pcol8 = iota & 7

    for c in range(_NCH):
        base = wid * _BPW + c * _CHUNK
        copies = [
            pltpu.async_copy(bigtabs[k].at[idx_v.at[s, c]], bufs[k], sem)
            for k, (s, col, w) in enumerate(_BIG)
        ]

        def small_group(g, carry):
            rb = g * 16 * _D
            for s, col, w, off in _SMALL:
                idxv = idx_v[s, c, pl.ds(g * 16, 16)]
                rowbase = (idxv << 3) + (off * 8)
                for k in range(w):
                    vals = plsc.load_gather(small_v, [rowbase + k])
                    plsc.store_scatter(stage, [rowpat + (rb + col + k)], vals)
            return carry

        lax.fori_loop(0, _CHUNK // 16, small_group, 0, unroll=False)

        for cp in copies:
            cp.wait()

        def row16(j, carry):
            v = bufs[0][j, :]
            plsc.store_scatter(stage, [iota + (j * _D + 40)], v)
            return carry

        lax.fori_loop(0, _CHUNK, row16, 0, unroll=4)

        def pair8(j, carry):
            rows = prow16 + (j * 2)
            dbase = pat8 + (j * 2 * _D)
            for k, (s, col, w) in enumerate(_BIG[1:], start=1):
                v = plsc.load_gather(bufs[k], [rows, pcol8])
                plsc.store_scatter(stage, [dbase + col], v)
            return carry

        lax.fori_loop(0, _CHUNK // 2, pair8, 0, unroll=2)

        pltpu.sync_copy(stage, out_hbm.at[pl.ds(base * _D, _CHUNK * _D)])


@jax.jit
def kernel(city_idx, W_city, state_idx, W_state, store_type_idx, W_store_type,
           cluster_idx, W_cluster, store_nbr_idx, W_store_nbr, family_idx,
           W_family, store_family_interaction_idx, W_store_family_interaction,
           onpromo_promo_sum7_interaction_idx, W_onpromo_promo_sum7_interaction,
           onpromo_state_interaction_idx, W_onpromo_state_interaction,
           promo_sum7_state_interaction_idx, W_promo_sum7_state_interaction):
    idxs = [city_idx, state_idx, store_type_idx, cluster_idx, store_nbr_idx,
            family_idx, store_family_interaction_idx,
            onpromo_promo_sum7_interaction_idx, onpromo_state_interaction_idx,
            promo_sum7_state_interaction_idx]
    idxs = [i.reshape(_NW, _NCH, _CHUNK) for i in idxs]
    smalls = [W_city, W_state, W_store_type, W_cluster, W_store_nbr, W_family]
    small_tab = jnp.concatenate(
        [jnp.pad(t, ((0, 0), (0, 8 - t.shape[1]))) for t in smalls], axis=0
    ).reshape(-1)
    bigtabs = [W_store_family_interaction, W_onpromo_promo_sum7_interaction,
               W_onpromo_state_interaction, W_promo_sum7_state_interaction]

    mesh = plsc.VectorSubcoreMesh(core_axis_name="c", subcore_axis_name="s")

    compact = functools.partial(
        pl.kernel,
        mesh=mesh,
        out_type=[jax.ShapeDtypeStruct((_NPAD * 16,), jnp.float32)] +
                 [jax.ShapeDtypeStruct((_NPAD * 8,), jnp.float32)] * 3,
        scratch_types=[
            pltpu.VMEM((_CROWS, 16), jnp.float32),
            pltpu.VMEM((_CROWS, 16), jnp.float32),
            pltpu.VMEM((_CROWS, 16), jnp.float32),
            pltpu.VMEM((_CROWS, 8), jnp.float32),
            pltpu.VMEM((_CROWS, 8), jnp.float32),
            pltpu.VMEM((_CROWS, 8), jnp.float32),
            pltpu.VMEM((_CROWS * 16,), jnp.float32),
            pltpu.VMEM((_CROWS * 16,), jnp.float32),
            pltpu.VMEM((1, 16), jnp.float32),
            pltpu.VMEM((1, 8), jnp.float32),
            pltpu.SemaphoreType.DMA,
            pltpu.SemaphoreType.DMA,
        ],
        compiler_params=pltpu.CompilerParams(
            use_tc_tiling_on_sc=True, needs_layout_passes=False),
    )(_compact_body)
    flat16, flat8a, flat8b, flat8c = compact(*bigtabs)
    ctabs = [flat16.reshape(_NPAD, 16), flat8a.reshape(_NPAD, 8),
             flat8b.reshape(_NPAD, 8), flat8c.reshape(_NPAD, 8)]

    run = functools.partial(
        pl.kernel,
        mesh=mesh,
        out_type=jax.ShapeDtypeStruct((_B * _D,), jnp.float32),
        scratch_types=[
            pltpu.VMEM((10, _NCH, _CHUNK), jnp.int32),
            pltpu.VMEM((_SMALL_WORDS,), jnp.float32),
            pltpu.VMEM((_CHUNK, 16), jnp.float32),
            pltpu.VMEM((_CHUNK, 8), jnp.float32),
            pltpu.VMEM((_CHUNK, 8), jnp.float32),
            pltpu.VMEM((_CHUNK, 8), jnp.float32),
            pltpu.VMEM((_CHUNK * _D,), jnp.float32),
            pltpu.SemaphoreType.DMA,
        ],
        compiler_params=pltpu.CompilerParams(
            use_tc_tiling_on_sc=False, needs_layout_passes=False),
    )(_gather_body)
    out = run(*idxs, small_tab, *ctabs)
    return out.reshape(_B, _D)
